# Initial kernel scaffold; baseline (speedup 1.0000x reference)
import jax
import jax.numpy as jnp
from jax.experimental import pallas as pl


def kernel(abs_embedding, coordinates, y):
    raise NotImplementedError("write your pallas kernel here")


# speedup vs baseline: 2.6003x; 2.6003x over previous
import functools

import jax
import jax.numpy as jnp
from jax import lax
from jax.experimental import pallas as pl
from jax.experimental.pallas import tpu as pltpu
from jax.experimental.pallas import tpu_sc as plsc

PUSH_MARGIN = 1.0
NUM_CLASSES = 8
B, N, C, H, W = 4, 4096, 32, 512, 512
PTS = B * N
NC, NS, L = 2, 16, 16
NW = NC * NS
PER_W = PTS // NW
CHW = 128
NCH = PER_W // CHW

@functools.cache
def _gather_labels_kernel():
    mesh = plsc.VectorSubcoreMesh(
        core_axis_name="c", subcore_axis_name="s", num_cores=NC, num_subcores=NS
    )

    @functools.partial(
        pl.kernel,
        out_type=jax.ShapeDtypeStruct((PTS,), jnp.int32),
        mesh=mesh,
        scratch_types=[
            pltpu.VMEM((PER_W,), jnp.int32),
            pltpu.VMEM((PER_W,), jnp.int32),
            pltpu.VMEM((NCH, CHW), jnp.int32),
            pltpu.VMEM((CHW,), jnp.int32),
            pltpu.SemaphoreType.DMA,
        ],
    )
    def _gather_labels(cx_hbm, cy_hbm, y_hbm, out_hbm, cx_v, cy_v, idx_v, lab_v, sem):
        wid = lax.axis_index("s") * NC + lax.axis_index("c")
        base = wid * PER_W
        boff = (base // N) * (H * W)
        pltpu.sync_copy(cx_hbm.at[pl.ds(base, PER_W)], cx_v)
        pltpu.sync_copy(cy_hbm.at[pl.ds(base, PER_W)], cy_v)
        for j in range(NCH):
            for t in range(CHW // L):
                s = j * CHW + t * L
                cx = cx_v[pl.ds(s, L)]
                cy = cy_v[pl.ds(s, L)]
                idx_v[j, pl.ds(t * L, L)] = cx * W + cy + boff
        for j in range(NCH):
            pltpu.async_copy(y_hbm.at[idx_v.at[j]], lab_v, sem).wait()
            pltpu.sync_copy(lab_v, out_hbm.at[pl.ds(base + j * CHW, CHW)])

    return _gather_labels


def _loss_kernel(emb_ref, lab_ref, out_ref):
    dn_seg = (((0,), (0,)), ((), ()))
    dn_mm = (((1,), (0,)), ((), ()))
    cls_row = lax.broadcasted_iota(jnp.int32, (1, NUM_CLASSES), 1)
    fg_row = (cls_row >= 1).astype(jnp.float32)
    ones_col = jnp.ones((N, 1), jnp.float32)
    rr = lax.broadcasted_iota(jnp.int32, (NUM_CLASSES, NUM_CLASSES), 0)
    cc = lax.broadcasted_iota(jnp.int32, (NUM_CLASSES, NUM_CLASSES), 1)
    upper = ((cc > rr) & (rr >= 1)).astype(jnp.float32)

    total = jnp.float32(0.0)
    for b in range(B):
        emb = emb_ref[b]
        lab = lab_ref[b].reshape(N, 1)
        oh = (lab == cls_row).astype(jnp.float32)
        counts_row = jnp.sum(oh, axis=0, keepdims=True)
        safe_row = jnp.maximum(counts_row, 1.0)
        sums = lax.dot_general(oh, emb, dn_seg,
                               precision=lax.Precision.HIGHEST,
                               preferred_element_type=jnp.float32)
        counts_col = lax.dot_general(oh, ones_col, dn_seg,
                                     precision=lax.Precision.HIGHEST,
                                     preferred_element_type=jnp.float32)
        cents = sums / jnp.maximum(counts_col, 1.0)
        cpp = lax.dot_general(oh, cents, dn_mm,
                              precision=lax.Precision.HIGHEST,
                              preferred_element_type=jnp.float32)
        diff = emb - cpp
        d = jnp.sqrt(jnp.sum(diff * diff, axis=1, keepdims=True))
        pull_sums = jnp.sum(oh * d, axis=0, keepdims=True)
        pull_c = pull_sums / safe_row
        presf_row = (counts_row > 0.0).astype(jnp.float32) * fg_row
        pull_over = jnp.sum(pull_c * presf_row)
        k = jnp.sum(presf_row)

        ca = lax.broadcast_in_dim(cents, (NUM_CLASSES, NUM_CLASSES, C), (0, 2))
        cb = lax.broadcast_in_dim(cents, (NUM_CLASSES, NUM_CLASSES, C), (1, 2))
        pd = jnp.sqrt(jnp.sum((ca - cb) ** 2, axis=2))
        presf_col = (counts_col > 0.0).astype(jnp.float32)
        pairm = presf_col * presf_row * upper
        n_pairs = jnp.sum(pairm)
        push_sum = jnp.sum(jnp.maximum(PUSH_MARGIN - pd, 0.0) * pairm)
        push_term = push_sum / jnp.maximum(n_pairs, 1.0)

        multi = k > 1.0
        contrib = pull_over / jnp.maximum(k, 1.0) + push_term
        total = total + jnp.where(multi, contrib, 0.0)
    out_ref[0, 0] = total


def _loss_from_labels(abs_embedding, labels):
    return pl.pallas_call(
        _loss_kernel,
        out_shape=jax.ShapeDtypeStruct((1, 1), jnp.float32),
        out_specs=pl.BlockSpec(memory_space=pltpu.SMEM),
    )(abs_embedding, labels)


def kernel(abs_embedding, coordinates, y):
    cx = coordinates[..., 1].reshape(-1).astype(jnp.int32)
    cy = coordinates[..., 0].reshape(-1).astype(jnp.int32)
    y_flat = y.reshape(-1).astype(jnp.int32)
    labels = _gather_labels_kernel()(cx, cy, y_flat)
    loss = _loss_from_labels(abs_embedding, labels.reshape(B, N))
    return loss[0, 0]


# speedup vs baseline: 3.6904x; 1.4192x over previous
import functools

import jax
import jax.numpy as jnp
from jax import lax
from jax.experimental import pallas as pl
from jax.experimental.pallas import tpu as pltpu
from jax.experimental.pallas import tpu_sc as plsc

PUSH_MARGIN = 1.0
NUM_CLASSES = 8
B, N, C, H, W = 4, 4096, 32, 512, 512
PTS = B * N
NC, NS, L = 2, 16, 16
NW = NC * NS
PER_W = PTS // NW
CHW = 128
NCH = PER_W // CHW


@functools.cache
def _gather_labels_kernel():
    mesh = plsc.VectorSubcoreMesh(
        core_axis_name="c", subcore_axis_name="s", num_cores=NC, num_subcores=NS
    )

    @functools.partial(
        pl.kernel,
        out_type=jax.ShapeDtypeStruct((PTS,), jnp.int32),
        mesh=mesh,
        scratch_types=[
            pltpu.VMEM((PER_W,), jnp.int32),
            pltpu.VMEM((PER_W,), jnp.int32),
            pltpu.VMEM((NCH, CHW), jnp.int32),
            pltpu.VMEM((NCH, CHW), jnp.int32),
            pltpu.SemaphoreType.DMA,
        ],
    )
    def _gather_labels(coords_hbm, y_hbm, out_hbm, cy_v, cx_v, idx_v, lab_v, sem):
        wid = lax.axis_index("s") * NC + lax.axis_index("c")
        base = wid * PER_W
        boff = (base // N) * (H * W)
        pltpu.sync_copy(coords_hbm.at[pl.ds(base, PER_W)], cy_v)
        pltpu.sync_copy(coords_hbm.at[pl.ds(PTS + base, PER_W)], cx_v)
        for j in range(NCH):
            for t in range(CHW // L):
                s = j * CHW + t * L
                cy = cy_v[pl.ds(s, L)]
                cx = cx_v[pl.ds(s, L)]
                idx_v[j, pl.ds(t * L, L)] = cx * W + cy + boff
        copies = [
            pltpu.async_copy(y_hbm.at[idx_v.at[j]], lab_v.at[j], sem)
            for j in range(NCH)
        ]
        for cp in copies:
            cp.wait()
        for j in range(NCH):
            pltpu.sync_copy(lab_v.at[j], out_hbm.at[pl.ds(base + j * CHW, CHW)])

    return _gather_labels


def _loss_kernel(embt_ref, lab_ref, out_ref):
    dn_ss = (((1,), (1,)), ((), ()))
    dn_mm = (((1,), (0,)), ((), ()))
    prec = lax.Precision.HIGHEST
    cls_col = lax.broadcasted_iota(jnp.int32, (NUM_CLASSES, 1), 0)
    fg_col = (cls_col >= 1).astype(jnp.float32)
    rr = lax.broadcasted_iota(jnp.int32, (NUM_CLASSES, NUM_CLASSES), 0)
    cc = lax.broadcasted_iota(jnp.int32, (NUM_CLASSES, NUM_CLASSES), 1)
    upper = ((cc > rr) & (rr >= 1)).astype(jnp.float32)

    total = jnp.float32(0.0)
    for b in range(B):
        embt = embt_ref[b]
        lab = lab_ref[b].reshape(1, N)
        oh = (lab == cls_col).astype(jnp.float32)
        counts_col = jnp.sum(oh, axis=1, keepdims=True)
        safe_col = jnp.maximum(counts_col, 1.0)
        safe_row = lax.transpose(safe_col, (1, 0))
        sums_t = lax.dot_general(embt, oh, dn_ss, precision=prec,
                                 preferred_element_type=jnp.float32)
        cents_t = sums_t / safe_row
        cpp_t = lax.dot_general(cents_t, oh, dn_mm, precision=prec,
                                preferred_element_type=jnp.float32)
        diff = embt - cpp_t
        d = jnp.sqrt(jnp.sum(diff * diff, axis=0, keepdims=True))
        pull_sums = jnp.sum(oh * d, axis=1, keepdims=True)
        pull_c = pull_sums / safe_col
        presf_col = (counts_col > 0.0).astype(jnp.float32) * fg_col
        pull_over = jnp.sum(pull_c * presf_col)
        k = jnp.sum(presf_col)

        gram = lax.dot_general(cents_t, cents_t, (((0,), (0,)), ((), ())),
                               precision=lax.Precision.HIGHEST,
                               preferred_element_type=jnp.float32)
        n2_row = jnp.sum(cents_t * cents_t, axis=0, keepdims=True)
        n2_col = lax.transpose(n2_row, (1, 0))
        pd2 = jnp.maximum(n2_row + n2_col - 2.0 * gram, 0.0)
        pd = jnp.sqrt(pd2)
        presf_row = lax.transpose(presf_col, (1, 0))
        pairm = presf_col * presf_row * upper
        n_pairs = jnp.sum(pairm)
        push_sum = jnp.sum(jnp.maximum(PUSH_MARGIN - pd, 0.0) * pairm)
        push_term = push_sum / jnp.maximum(n_pairs, 1.0)

        multi = k > 1.0
        contrib = pull_over / jnp.maximum(k, 1.0) + push_term
        total = total + jnp.where(multi, contrib, 0.0)
    out_ref[0, 0] = total


def _loss_from_labels(embt, labels):
    return pl.pallas_call(
        _loss_kernel,
        out_shape=jax.ShapeDtypeStruct((1, 1), jnp.float32),
        out_specs=pl.BlockSpec(memory_space=pltpu.SMEM),
    )(embt, labels)


def kernel(abs_embedding, coordinates, y):
    coords_t = jnp.transpose(coordinates.astype(jnp.int32), (2, 0, 1)).reshape(-1)
    y_flat = y.reshape(-1).astype(jnp.int32)
    labels = _gather_labels_kernel()(coords_t, y_flat)
    embt = jnp.transpose(abs_embedding, (0, 2, 1))
    loss = _loss_from_labels(embt, labels.reshape(B, N))
    return loss[0, 0]


# speedup vs baseline: 4.4681x; 1.2107x over previous
import functools

import jax
import jax.numpy as jnp
from jax import lax
from jax.experimental import pallas as pl
from jax.experimental.pallas import tpu as pltpu
from jax.experimental.pallas import tpu_sc as plsc

PUSH_MARGIN = 1.0
NUM_CLASSES = 8
B, N, C, H, W = 4, 4096, 32, 512, 512
PTS = B * N
NC, NS, L = 2, 16, 16
NW = NC * NS
PER_W = PTS // NW
CHW = 128
NCH = PER_W // CHW


@functools.cache
def _gather_labels_kernel():
    mesh = plsc.VectorSubcoreMesh(
        core_axis_name="c", subcore_axis_name="s", num_cores=NC, num_subcores=NS
    )

    @functools.partial(
        pl.kernel,
        out_type=jax.ShapeDtypeStruct((B, N), jnp.int32),
        mesh=mesh,
        scratch_types=[
            pltpu.VMEM((PER_W,), jnp.int32),
            pltpu.VMEM((PER_W,), jnp.int32),
            pltpu.VMEM((NCH, CHW), jnp.int32),
            pltpu.VMEM((NCH, CHW), jnp.int32),
            pltpu.SemaphoreType.DMA,
        ],
    )
    def _gather_labels(coords_hbm, y_hbm, out_hbm, cy_v, cx_v, idx_v, lab_v, sem):
        wid = lax.axis_index("s") * NC + lax.axis_index("c")
        base = wid * PER_W
        boff = (base // N) * (H * W)
        pltpu.sync_copy(coords_hbm.at[pl.ds(base, PER_W)], cy_v)
        pltpu.sync_copy(coords_hbm.at[pl.ds(PTS + base, PER_W)], cx_v)
        for j in range(NCH):
            for t in range(CHW // L):
                s = j * CHW + t * L
                cy = cy_v[pl.ds(s, L)]
                cx = cx_v[pl.ds(s, L)]
                idx_v[j, pl.ds(t * L, L)] = cx * W + cy + boff
        copies = [
            pltpu.async_copy(y_hbm.at[idx_v.at[j]], lab_v.at[j], sem)
            for j in range(NCH)
        ]
        for cp in copies:
            cp.wait()
        bi = base // N
        inb = base % N
        for j in range(NCH):
            pltpu.sync_copy(lab_v.at[j], out_hbm.at[bi, pl.ds(inb + j * CHW, CHW)])

    return _gather_labels


def _loss_kernel(embt_ref, lab_ref, out_ref):
    dn_ss = (((1,), (1,)), ((), ()))
    dn_mm = (((1,), (0,)), ((), ()))
    prec = lax.Precision.DEFAULT
    cls_col = lax.broadcasted_iota(jnp.int32, (NUM_CLASSES, 1), 0)
    fg_col = (cls_col >= 1).astype(jnp.float32)
    rr = lax.broadcasted_iota(jnp.int32, (NUM_CLASSES, NUM_CLASSES), 0)
    cc = lax.broadcasted_iota(jnp.int32, (NUM_CLASSES, NUM_CLASSES), 1)
    upper = ((cc > rr) & (rr >= 1)).astype(jnp.float32)

    total = jnp.float32(0.0)
    for b in range(B):
        embt = embt_ref[b]
        lab = lab_ref[b].reshape(1, N)
        oh = (lab == cls_col).astype(jnp.float32)
        counts_col = jnp.sum(oh, axis=1, keepdims=True)
        safe_col = jnp.maximum(counts_col, 1.0)
        safe_row = lax.transpose(safe_col, (1, 0))
        sums_t = lax.dot_general(embt, oh, dn_ss, precision=prec,
                                 preferred_element_type=jnp.float32)
        cents_t = sums_t / safe_row
        cpp_t = lax.dot_general(cents_t, oh, dn_mm, precision=prec,
                                preferred_element_type=jnp.float32)
        diff = embt - cpp_t
        d = jnp.sqrt(jnp.sum(diff * diff, axis=0, keepdims=True))
        pull_sums = jnp.sum(oh * d, axis=1, keepdims=True)
        pull_c = pull_sums / safe_col
        presf_col = (counts_col > 0.0).astype(jnp.float32) * fg_col
        pull_over = jnp.sum(pull_c * presf_col)
        k = jnp.sum(presf_col)

        gram = lax.dot_general(cents_t, cents_t, (((0,), (0,)), ((), ())),
                               precision=lax.Precision.HIGHEST,
                               preferred_element_type=jnp.float32)
        n2_row = jnp.sum(cents_t * cents_t, axis=0, keepdims=True)
        n2_col = lax.transpose(n2_row, (1, 0))
        pd2 = jnp.maximum(n2_row + n2_col - 2.0 * gram, 0.0)
        pd = jnp.sqrt(pd2)
        presf_row = lax.transpose(presf_col, (1, 0))
        pairm = presf_col * presf_row * upper
        n_pairs = jnp.sum(pairm)
        push_sum = jnp.sum(jnp.maximum(PUSH_MARGIN - pd, 0.0) * pairm)
        push_term = push_sum / jnp.maximum(n_pairs, 1.0)

        multi = k > 1.0
        contrib = pull_over / jnp.maximum(k, 1.0) + push_term
        total = total + jnp.where(multi, contrib, 0.0)
    out_ref[0, 0] = total


def _loss_from_labels(embt, labels):
    return pl.pallas_call(
        _loss_kernel,
        out_shape=jax.ShapeDtypeStruct((1, 1), jnp.float32),
        out_specs=pl.BlockSpec(memory_space=pltpu.SMEM),
    )(embt, labels)


def kernel(abs_embedding, coordinates, y):
    coords_t = jnp.transpose(coordinates.astype(jnp.int32), (2, 0, 1)).reshape(-1)
    y_flat = y.reshape(-1).astype(jnp.int32)
    labels = _gather_labels_kernel()(coords_t, y_flat)
    embt = jnp.transpose(abs_embedding, (0, 2, 1))
    loss = _loss_from_labels(embt, labels)
    return loss[0, 0]
